# baseline (speedup 1.0000x reference)
import functools

import jax
import jax.numpy as jnp
from jax import lax
from jax.experimental import pallas as pl
from jax.experimental.pallas import tpu as pltpu
from jax.experimental.pallas import tpu_sc as plsc

_VOCAB = 1000
_EMBED = 64
_B = 4096
_L = 200
_N = _B * _L

_NC = 2
_NS = 16
_NW = _NC * _NS
_PER_W = _N // _NW
_CHUNK = 256
_NBUF = 4
_STEPS = _PER_W // _CHUNK
_GROUPS = _STEPS // _NBUF


@functools.partial(
    pl.kernel,
    mesh=plsc.VectorSubcoreMesh(core_axis_name="c", subcore_axis_name="s"),
    out_type=jax.ShapeDtypeStruct((_N, _EMBED), jnp.float32),
    scratch_types=[
        pltpu.VMEM((_STEPS, _CHUNK), jnp.int32),
        pltpu.VMEM((_NBUF, _CHUNK, _EMBED), jnp.float32),
        [pltpu.SemaphoreType.DMA] * _NBUF,
        [pltpu.SemaphoreType.DMA] * _NBUF,
    ],
    compiler_params=pltpu.CompilerParams(use_tc_tiling_on_sc=False),
)
def _gather_kernel(idx_hbm, table_hbm, out_hbm, idx_v, rows_v, gsems, ssems):
    wid = lax.axis_index("s") * _NC + lax.axis_index("c")
    base = wid * _PER_W

    pltpu.sync_copy(idx_hbm.at[wid], idx_v)

    def fire_gather(i, b):
        return pltpu.async_copy(table_hbm.at[idx_v.at[i]], rows_v.at[b],
                                gsems[b])

    def fire_store(i, b):
        off = base + i * _CHUNK
        return pltpu.async_copy(rows_v.at[b],
                                out_hbm.at[pl.ds(off, _CHUNK)], ssems[b])

    gd = [fire_gather(b, b) for b in range(_NBUF)]
    sd = [pltpu.make_async_copy(rows_v.at[b],
                                out_hbm.at[pl.ds(base, _CHUNK)], ssems[b])
          for b in range(_NBUF)]

    def body(j, carry):
        for b in range(_NBUF):
            i = j * _NBUF + b
            gd[b].wait()
            fire_store(i, b)
            sd[b].wait()
            fire_gather(i + _NBUF, b)
        return carry

    lax.fori_loop(0, _GROUPS - 1, body, 0)

    for b in range(_NBUF):
        i = (_GROUPS - 1) * _NBUF + b
        gd[b].wait()
        fire_store(i, b)
    for b in range(_NBUF):
        sd[b].wait()


def kernel(seqs, att_mask, word_embedding):
    del att_mask
    idx = seqs.reshape(_NW, _STEPS, _CHUNK)
    out = _gather_kernel(idx, word_embedding)
    return out.reshape(_B, _L, _EMBED)


# speedup vs baseline: 1.4074x; 1.4074x over previous
import functools

import jax
import jax.numpy as jnp
from jax import lax
from jax.experimental import pallas as pl
from jax.experimental.pallas import tpu as pltpu
from jax.experimental.pallas import tpu_sc as plsc

_VOCAB = 1000
_EMBED = 64
_B = 4096
_L = 200
_N = _B * _L

_NC = 2
_NS = 16
_NW = _NC * _NS
_PER_W = _N // _NW
_CHUNK = 256
_NBUF = 4
_STEPS = _PER_W // _CHUNK
_GROUPS = _STEPS // _NBUF


@functools.partial(
    pl.kernel,
    mesh=plsc.VectorSubcoreMesh(core_axis_name="c", subcore_axis_name="s"),
    out_type=jax.ShapeDtypeStruct((_N, _EMBED), jnp.float32),
    scratch_types=[
        pltpu.VMEM((_STEPS, _CHUNK), jnp.int32),
        pltpu.VMEM((_NBUF, _CHUNK, _EMBED), jnp.float32),
        pltpu.VMEM_SHARED((_VOCAB, _EMBED), jnp.float32),
        [pltpu.SemaphoreType.DMA] * _NBUF,
        [pltpu.SemaphoreType.DMA] * _NBUF,
    ],
    compiler_params=pltpu.CompilerParams(use_tc_tiling_on_sc=False),
)
def _gather_kernel(idx_hbm, table_hbm, out_hbm, idx_v, rows_v, table_sh,
                   gsems, ssems):
    s = lax.axis_index("s")
    wid = s * _NC + lax.axis_index("c")
    base = wid * _PER_W

    @pl.when(s == 0)
    def _():
        pltpu.sync_copy(table_hbm, table_sh)

    pltpu.sync_copy(idx_hbm.at[wid], idx_v)
    plsc.subcore_barrier()

    def fire_gather(i, b):
        return pltpu.async_copy(table_sh.at[idx_v.at[i]], rows_v.at[b],
                                gsems[b])

    def fire_store(i, b):
        off = base + i * _CHUNK
        return pltpu.async_copy(rows_v.at[b],
                                out_hbm.at[pl.ds(off, _CHUNK)], ssems[b])

    gd = [fire_gather(b, b) for b in range(_NBUF)]
    sd = [pltpu.make_async_copy(rows_v.at[b],
                                out_hbm.at[pl.ds(base, _CHUNK)], ssems[b])
          for b in range(_NBUF)]

    def body(j, carry):
        for b in range(_NBUF):
            i = j * _NBUF + b
            gd[b].wait()
            fire_store(i, b)
            sd[b].wait()
            fire_gather(i + _NBUF, b)
        return carry

    lax.fori_loop(0, _GROUPS - 1, body, 0)

    for b in range(_NBUF):
        i = (_GROUPS - 1) * _NBUF + b
        gd[b].wait()
        fire_store(i, b)
    for b in range(_NBUF):
        sd[b].wait()


def kernel(seqs, att_mask, word_embedding):
    del att_mask
    idx = seqs.reshape(_NW, _STEPS, _CHUNK)
    out = _gather_kernel(idx, word_embedding)
    return out.reshape(_B, _L, _EMBED)


# speedup vs baseline: 1.4108x; 1.0024x over previous
import functools

import jax
import jax.numpy as jnp
from jax import lax
from jax.experimental import pallas as pl
from jax.experimental.pallas import tpu as pltpu
from jax.experimental.pallas import tpu_sc as plsc

_VOCAB = 1000
_EMBED = 64
_B = 4096
_L = 200

_NC = 2
_NS = 16
_NW = _NC * _NS
_PER_W = _B // _NW
_NBUF = 4
_GROUPS = _PER_W // _NBUF


@functools.partial(
    pl.kernel,
    mesh=plsc.VectorSubcoreMesh(core_axis_name="c", subcore_axis_name="s"),
    out_type=jax.ShapeDtypeStruct((_B, _L, _EMBED), jnp.float32),
    scratch_types=[
        pltpu.VMEM((_PER_W, _L), jnp.int32),
        pltpu.VMEM((_NBUF, _L, _EMBED), jnp.float32),
        pltpu.VMEM_SHARED((_VOCAB, _EMBED), jnp.float32),
        [pltpu.SemaphoreType.DMA] * _NBUF,
        [pltpu.SemaphoreType.DMA] * _NBUF,
    ],
    compiler_params=pltpu.CompilerParams(use_tc_tiling_on_sc=False),
)
def _gather_kernel(idx_hbm, table_hbm, out_hbm, idx_v, rows_v, table_sh,
                   gsems, ssems):
    s = lax.axis_index("s")
    wid = s * _NC + lax.axis_index("c")
    base = wid * _PER_W

    @pl.when(s == 0)
    def _():
        pltpu.sync_copy(table_hbm, table_sh)

    pltpu.sync_copy(idx_hbm.at[wid], idx_v)
    plsc.subcore_barrier()

    def fire_gather(i, b):
        return pltpu.async_copy(table_sh.at[idx_v.at[i]], rows_v.at[b],
                                gsems[b])

    def fire_store(i, b):
        return pltpu.async_copy(rows_v.at[b], out_hbm.at[base + i], ssems[b])

    gd = [fire_gather(b, b) for b in range(_NBUF)]
    sd = [pltpu.make_async_copy(rows_v.at[b], out_hbm.at[base], ssems[b])
          for b in range(_NBUF)]

    def body(j, carry):
        for b in range(_NBUF):
            i = j * _NBUF + b
            gd[b].wait()
            fire_store(i, b)
            sd[b].wait()
            fire_gather(i + _NBUF, b)
        return carry

    lax.fori_loop(0, _GROUPS - 1, body, 0)

    for b in range(_NBUF):
        i = (_GROUPS - 1) * _NBUF + b
        gd[b].wait()
        fire_store(i, b)
    for b in range(_NBUF):
        sd[b].wait()


def kernel(seqs, att_mask, word_embedding):
    del att_mask
    idx = seqs.reshape(_NW, _PER_W, _L)
    return _gather_kernel(idx, word_embedding)
